# baseline (speedup 1.0000x reference)
import functools
import math

import jax
import jax.numpy as jnp
from jax import lax
from jax.experimental import pallas as pl
from jax.experimental.pallas import tpu as pltpu
from jax.experimental.pallas import tpu_sc as plsc

_NPTS = 20000
_NPAD = 20480
_CHUNK = 2048
_NC = 1024
_NBACK = 10000
_NBACK_PAD = 10240
_NES = 2048
_NGT = 4096
_MAXPTS = 256.0


def _argmin_kernel(q_ref, pt_ref, out_ref):
    q = q_ref[0]
    sq_q = jnp.sum(q * q, axis=1, keepdims=True)
    qm2 = -2.0 * q

    m_run = jnp.full((_NC, _CHUNK), jnp.inf, jnp.float32)
    c_run = jnp.zeros((_NC, _CHUNK), jnp.float32)
    for c in range(_NPAD // _CHUNK):
        chunk = pt_ref[0, :, c * _CHUNK:(c + 1) * _CHUNK]
        sq_p = jnp.sum(chunk * chunk, axis=0, keepdims=True)
        qp2 = jnp.dot(qm2, chunk, preferred_element_type=jnp.float32)
        d2 = (sq_q + sq_p) + qp2
        mask = d2 < m_run
        c_run = jnp.where(mask, float(c), c_run)
        m_run = jnp.where(mask, d2, m_run)
    gmin = jnp.min(m_run, axis=1, keepdims=True)
    iota_f = lax.broadcasted_iota(
        jnp.int32, (_NC, _CHUNK), 1).astype(jnp.float32)
    gidx = c_run * float(_CHUNK) + iota_f
    best = jnp.min(jnp.where(m_run == gmin, gidx, 3.0e7), axis=1,
                   keepdims=True)
    out_ref[0] = best.astype(jnp.int32)


def _nn_indices(queries, points_t):
    return pl.pallas_call(
        _argmin_kernel,
        grid=(2,),
        in_specs=[
            pl.BlockSpec((1, _NC, 3), lambda i: (i, 0, 0)),
            pl.BlockSpec((1, 3, _NPAD), lambda i: (i, 0, 0)),
        ],
        out_specs=pl.BlockSpec((1, _NC, 1), lambda i: (i, 0, 0)),
        out_shape=jax.ShapeDtypeStruct((2, _NC, 1), jnp.int32),
    )(queries, points_t)


def _sc_body(idx_hbm, back_hbm, gt_hbm, zeros_hbm, mask_hbm, gmask_hbm,
             bitmap_v, back_v, qidx_v, qout_v, gidx_v, gout_v):
    side = lax.axis_index("c")
    sid = lax.axis_index("s")

    @pl.when(sid == 0)
    def _():
        pltpu.sync_copy(zeros_hbm.at[side], bitmap_v)
        pltpu.sync_copy(back_hbm.at[side], back_v)
        pltpu.sync_copy(idx_hbm.at[side], qidx_v)
        pltpu.sync_copy(gt_hbm.at[side], gidx_v)

        ones = jnp.ones((16,), jnp.int32)

        for i in range(_NBACK_PAD // 16):
            iv = back_v[i * 16:(i + 1) * 16]
            plsc.store_scatter(bitmap_v, [iv], ones)

        for i in range(_NC // 16):
            iv = qidx_v[i * 16:(i + 1) * 16]
            qout_v[i * 16:(i + 1) * 16] = plsc.load_gather(bitmap_v, [iv])

        pltpu.sync_copy(qout_v, mask_hbm.at[side])

        for i in range(_NGT // 16):
            iv = gidx_v[i * 16:(i + 1) * 16]
            gout_v[i * 16:(i + 1) * 16] = plsc.load_gather(qout_v, [iv])

        pltpu.sync_copy(gout_v, gmask_hbm.at[side])


def _sc_masks(idx_all, back_all, gt_all, zeros_all):
    mesh = plsc.VectorSubcoreMesh(core_axis_name="c", subcore_axis_name="s")
    f = pl.kernel(
        _sc_body,
        out_type=[jax.ShapeDtypeStruct((2, _NC), jnp.int32),
                  jax.ShapeDtypeStruct((2, _NGT), jnp.int32)],
        mesh=mesh,
        compiler_params=pltpu.CompilerParams(needs_layout_passes=False),
        scratch_types=[
            pltpu.VMEM((_NPAD,), jnp.int32),
            pltpu.VMEM((_NBACK_PAD,), jnp.int32),
            pltpu.VMEM((_NC,), jnp.int32),
            pltpu.VMEM((_NC,), jnp.int32),
            pltpu.VMEM((_NGT,), jnp.int32),
            pltpu.VMEM((_NGT,), jnp.int32),
        ],
    )
    return f(idx_all, back_all, gt_all, zeros_all)


def _logic_kernel(msrc_in, mref_in, msrcc_in, mrefr_in,
                  esr_col, esc_col, esr_row, esc_row,
                  gtr_col, gtc_col, gtr_row, gtc_row,
                  ov_col, ov_row, out_ref):
    f32 = jnp.float32

    msrc_col = msrc_in[...] != 0
    mref_col = mref_in[...] != 0
    cnt_mask = (jnp.sum(msrc_col.astype(f32)) + jnp.sum(mref_col.astype(f32)))
    loss2 = (2.0 * _NC - cnt_mask) / (2.0 * _NC)

    esk_col = esr_col[...] * _NC + esc_col[...]
    esk_row = esr_row[...] * _NC + esc_row[...]
    i_es = lax.broadcasted_iota(jnp.int32, (_NES, _NES), 0)
    j_es = lax.broadcasted_iota(jnp.int32, (_NES, _NES), 1)
    dup_before = jnp.any((esk_col == esk_row) & (j_es < i_es), axis=1,
                         keepdims=True)
    e_cnt = _NES - jnp.sum(dup_before.astype(f32))

    gtk_col = gtr_col[...] * _NC + gtc_col[...]
    gtk_row = gtr_row[...] * _NC + gtc_row[...]

    not_rep_col = jnp.zeros((_NGT, 1), jnp.bool_)
    for c in range(4):
        kr = gtk_row[:, c * 1024:(c + 1) * 1024]
        i_i = lax.broadcasted_iota(jnp.int32, (_NGT, 1024), 0)
        j_i = lax.broadcasted_iota(jnp.int32, (_NGT, 1024), 1) + c * 1024
        later = jnp.any((gtk_col == kr) & (j_i > i_i), axis=1, keepdims=True)
        not_rep_col = jnp.logical_or(not_rep_col, later)

    mref_at_r_col = mrefr_in[...] != 0
    msrc_at_c_col = msrcc_in[...] != 0

    cand_col = (~not_rep_col) & mref_at_r_col & msrc_at_c_col
    cand_row = jnp.reshape(cand_col, (1, _NGT))
    nb = jnp.sum(cand_col.astype(f32))

    ov_row_v = ov_row[...]
    ovm_col = jnp.where(cand_col, ov_col[...], -1.0)
    rank_row = jnp.zeros((1, _NGT), f32)
    for c in range(4):
        sl = slice(c * 1024, (c + 1) * 1024)
        better = ovm_col[sl, :] > ov_row_v
        rank_row = rank_row + jnp.sum(better.astype(f32), axis=0,
                                      keepdims=True)
    sel_row = cand_row & (rank_row < _MAXPTS)

    es_member_row = jnp.zeros((1, _NGT), jnp.bool_)
    for c in range(2):
        ek = esk_col[c * 1024:(c + 1) * 1024, :]
        hit = jnp.any(ek == gtk_row, axis=0, keepdims=True)
        es_member_row = jnp.logical_or(es_member_row, hit)

    both_topk = jnp.sum((sel_row & es_member_row).astype(f32))
    both_plain = jnp.sum((cand_row & es_member_row).astype(f32))

    use_topk = nb > _MAXPTS
    g_cnt = jnp.where(use_topk, _MAXPTS, nb)
    both = jnp.where(use_topk, both_topk, both_plain)

    indices_f = jnp.maximum(e_cnt, 1.0)
    loss1 = math.sqrt(2.0) * (g_cnt + e_cnt - 2.0 * both) / indices_f
    loss = loss1 + loss2

    out_iota = lax.broadcasted_iota(jnp.int32, (1, 128), 1)
    out_ref[...] = jnp.where(out_iota == 0, loss,
                             jnp.where(out_iota == 1, loss1, loss2))


def _logic(args):
    return pl.pallas_call(
        _logic_kernel,
        out_shape=jax.ShapeDtypeStruct((1, 128), jnp.float32),
    )(*args)


def kernel(src_points, ref_points, src_points_c, ref_points_c,
           src_node_corr_indices, ref_node_corr_indices,
           gt_node_corr_indices, gt_node_corr_overlaps, transform,
           src_back_indices, ref_back_indices):
    del transform
    f32 = jnp.float32

    queries = jnp.stack([src_points_c, ref_points_c], axis=0)
    pts = jnp.stack([src_points, ref_points], axis=0)
    pts_pad = jnp.pad(pts, ((0, 0), (0, _NPAD - _NPTS), (0, 0)),
                      constant_values=1e9)
    points_t = jnp.transpose(pts_pad, (0, 2, 1))

    idx = _nn_indices(queries, points_t)
    idx_all = jnp.reshape(idx, (2, _NC))

    back_all = jnp.stack([
        jnp.pad(src_back_indices, (0, _NBACK_PAD - _NBACK),
                constant_values=_NPAD - 1),
        jnp.pad(ref_back_indices, (0, _NBACK_PAD - _NBACK),
                constant_values=_NPAD - 1)])
    gt_all = jnp.stack([gt_node_corr_indices[:, 1],
                        gt_node_corr_indices[:, 0]])
    zeros_all = jnp.zeros((2, _NPAD), jnp.int32)
    mask, gmask = _sc_masks(idx_all, back_all, gt_all, zeros_all)

    args = (
        mask[0].reshape(_NC, 1), mask[1].reshape(_NC, 1),
        gmask[0].reshape(_NGT, 1), gmask[1].reshape(_NGT, 1),
        ref_node_corr_indices.reshape(_NES, 1),
        src_node_corr_indices.reshape(_NES, 1),
        ref_node_corr_indices.reshape(1, _NES),
        src_node_corr_indices.reshape(1, _NES),
        gt_node_corr_indices[:, 0].reshape(_NGT, 1),
        gt_node_corr_indices[:, 1].reshape(_NGT, 1),
        gt_node_corr_indices[:, 0].reshape(1, _NGT),
        gt_node_corr_indices[:, 1].reshape(1, _NGT),
        gt_node_corr_overlaps.reshape(_NGT, 1).astype(f32),
        gt_node_corr_overlaps.reshape(1, _NGT).astype(f32),
    )
    out = _logic(args)
    loss = out[0, 0]
    loss1 = out[0, 1]
    loss2 = out[0, 2]
    return (loss, loss1, loss2)


# speedup vs baseline: 1.0768x; 1.0768x over previous
import functools
import math

import jax
import jax.numpy as jnp
from jax import lax
from jax.experimental import pallas as pl
from jax.experimental.pallas import tpu as pltpu
from jax.experimental.pallas import tpu_sc as plsc

_NPTS = 20000
_NPAD = 20480
_CHUNK = 2048
_NC = 1024
_NBACK = 10000
_NBACK_PAD = 10240
_NES = 2048
_NGT = 4096
_MAXPTS = 256.0


def _argmin_kernel(q_ref, pt_ref, out_ref):
    q = q_ref[0]
    sq_q = jnp.sum(q * q, axis=1, keepdims=True)
    qm2 = -2.0 * q

    best_val = jnp.full((_NC, 1), jnp.inf, jnp.float32)
    best_idx = jnp.full((_NC, 1), 0.0, jnp.float32)
    iota_f = lax.broadcasted_iota(
        jnp.int32, (_NC, _CHUNK), 1).astype(jnp.float32)
    for c in range(_NPAD // _CHUNK):
        chunk = pt_ref[0, :, c * _CHUNK:(c + 1) * _CHUNK]
        sq_p = jnp.sum(chunk * chunk, axis=0, keepdims=True)
        qp2 = jnp.dot(qm2, chunk, preferred_element_type=jnp.float32)
        d2 = (sq_q + sq_p) + qp2
        cmin = jnp.min(d2, axis=1, keepdims=True)
        carg = jnp.min(jnp.where(d2 == cmin, iota_f, 3.0e7), axis=1,
                       keepdims=True) + float(c * _CHUNK)
        better = cmin < best_val
        best_val = jnp.where(better, cmin, best_val)
        best_idx = jnp.where(better, carg, best_idx)
    out_ref[0] = best_idx.astype(jnp.int32)


def _nn_indices(queries, points_t):
    return pl.pallas_call(
        _argmin_kernel,
        grid=(2,),
        in_specs=[
            pl.BlockSpec((1, _NC, 3), lambda i: (i, 0, 0)),
            pl.BlockSpec((1, 3, _NPAD), lambda i: (i, 0, 0)),
        ],
        out_specs=pl.BlockSpec((1, _NC, 1), lambda i: (i, 0, 0)),
        out_shape=jax.ShapeDtypeStruct((2, _NC, 1), jnp.int32),
    )(queries, points_t)


def _sc_body(idx_hbm, back_hbm, gt_hbm, zeros_hbm, mask_hbm, gmask_hbm,
             bitmap_v, back_v, qidx_v, qout_v, gidx_v, gout_v):
    side = lax.axis_index("c")
    sid = lax.axis_index("s")

    @pl.when(sid == 0)
    def _():
        pltpu.sync_copy(zeros_hbm.at[side], bitmap_v)
        pltpu.sync_copy(back_hbm.at[side], back_v)
        pltpu.sync_copy(idx_hbm.at[side], qidx_v)
        pltpu.sync_copy(gt_hbm.at[side], gidx_v)

        ones = jnp.ones((16,), jnp.int32)

        for i in range(_NBACK_PAD // 16):
            iv = back_v[i * 16:(i + 1) * 16]
            plsc.store_scatter(bitmap_v, [iv], ones)

        for i in range(_NC // 16):
            iv = qidx_v[i * 16:(i + 1) * 16]
            qout_v[i * 16:(i + 1) * 16] = plsc.load_gather(bitmap_v, [iv])

        pltpu.sync_copy(qout_v, mask_hbm.at[side])

        for i in range(_NGT // 16):
            iv = gidx_v[i * 16:(i + 1) * 16]
            gout_v[i * 16:(i + 1) * 16] = plsc.load_gather(qout_v, [iv])

        pltpu.sync_copy(gout_v, gmask_hbm.at[side])


def _sc_masks(idx_all, back_all, gt_all, zeros_all):
    mesh = plsc.VectorSubcoreMesh(core_axis_name="c", subcore_axis_name="s")
    f = pl.kernel(
        _sc_body,
        out_type=[jax.ShapeDtypeStruct((2, _NC), jnp.int32),
                  jax.ShapeDtypeStruct((2, _NGT), jnp.int32)],
        mesh=mesh,
        compiler_params=pltpu.CompilerParams(needs_layout_passes=False),
        scratch_types=[
            pltpu.VMEM((_NPAD,), jnp.int32),
            pltpu.VMEM((_NBACK_PAD,), jnp.int32),
            pltpu.VMEM((_NC,), jnp.int32),
            pltpu.VMEM((_NC,), jnp.int32),
            pltpu.VMEM((_NGT,), jnp.int32),
            pltpu.VMEM((_NGT,), jnp.int32),
        ],
    )
    return f(idx_all, back_all, gt_all, zeros_all)


def _logic_kernel(msrc_in, mref_in, msrcc_in, mrefr_in,
                  esr_col, esc_col, esr_row, esc_row,
                  gtr_col, gtc_col, gtr_row, gtc_row,
                  ov_col, ov_row, out_ref):
    f32 = jnp.float32

    msrc_col = msrc_in[...] != 0
    mref_col = mref_in[...] != 0
    cnt_mask = (jnp.sum(msrc_col.astype(f32)) + jnp.sum(mref_col.astype(f32)))
    loss2 = (2.0 * _NC - cnt_mask) / (2.0 * _NC)

    esk_col = esr_col[...] * _NC + esc_col[...]
    esk_row = esr_row[...] * _NC + esc_row[...]
    i_es = lax.broadcasted_iota(jnp.int32, (_NES, _NES), 0)
    j_es = lax.broadcasted_iota(jnp.int32, (_NES, _NES), 1)
    dup_before = jnp.any((esk_col == esk_row) & (j_es < i_es), axis=1,
                         keepdims=True)
    e_cnt = _NES - jnp.sum(dup_before.astype(f32))

    gtk_col = gtr_col[...] * _NC + gtc_col[...]
    gtk_row = gtr_row[...] * _NC + gtc_row[...]

    not_rep_col = jnp.zeros((_NGT, 1), jnp.bool_)
    for c in range(4):
        kr = gtk_row[:, c * 1024:(c + 1) * 1024]
        i_i = lax.broadcasted_iota(jnp.int32, (_NGT, 1024), 0)
        j_i = lax.broadcasted_iota(jnp.int32, (_NGT, 1024), 1) + c * 1024
        later = jnp.any((gtk_col == kr) & (j_i > i_i), axis=1, keepdims=True)
        not_rep_col = jnp.logical_or(not_rep_col, later)

    mref_at_r_col = mrefr_in[...] != 0
    msrc_at_c_col = msrcc_in[...] != 0

    cand_col = (~not_rep_col) & mref_at_r_col & msrc_at_c_col
    cand_row = jnp.reshape(cand_col, (1, _NGT))
    nb = jnp.sum(cand_col.astype(f32))

    ov_row_v = ov_row[...]
    ovm_col = jnp.where(cand_col, ov_col[...], -1.0)
    rank_row = jnp.zeros((1, _NGT), f32)
    for c in range(4):
        sl = slice(c * 1024, (c + 1) * 1024)
        better = ovm_col[sl, :] > ov_row_v
        rank_row = rank_row + jnp.sum(better.astype(f32), axis=0,
                                      keepdims=True)
    sel_row = cand_row & (rank_row < _MAXPTS)

    es_member_row = jnp.zeros((1, _NGT), jnp.bool_)
    for c in range(2):
        ek = esk_col[c * 1024:(c + 1) * 1024, :]
        hit = jnp.any(ek == gtk_row, axis=0, keepdims=True)
        es_member_row = jnp.logical_or(es_member_row, hit)

    both_topk = jnp.sum((sel_row & es_member_row).astype(f32))
    both_plain = jnp.sum((cand_row & es_member_row).astype(f32))

    use_topk = nb > _MAXPTS
    g_cnt = jnp.where(use_topk, _MAXPTS, nb)
    both = jnp.where(use_topk, both_topk, both_plain)

    indices_f = jnp.maximum(e_cnt, 1.0)
    loss1 = math.sqrt(2.0) * (g_cnt + e_cnt - 2.0 * both) / indices_f
    loss = loss1 + loss2

    out_iota = lax.broadcasted_iota(jnp.int32, (1, 128), 1)
    out_ref[...] = jnp.where(out_iota == 0, loss,
                             jnp.where(out_iota == 1, loss1, loss2))


def _logic(args):
    return pl.pallas_call(
        _logic_kernel,
        out_shape=jax.ShapeDtypeStruct((1, 128), jnp.float32),
    )(*args)


def kernel(src_points, ref_points, src_points_c, ref_points_c,
           src_node_corr_indices, ref_node_corr_indices,
           gt_node_corr_indices, gt_node_corr_overlaps, transform,
           src_back_indices, ref_back_indices):
    del transform
    f32 = jnp.float32

    queries = jnp.stack([src_points_c, ref_points_c], axis=0)
    pts = jnp.stack([src_points, ref_points], axis=0)
    pts_pad = jnp.pad(pts, ((0, 0), (0, _NPAD - _NPTS), (0, 0)),
                      constant_values=1e9)
    points_t = jnp.transpose(pts_pad, (0, 2, 1))

    idx = _nn_indices(queries, points_t)
    idx_all = jnp.reshape(idx, (2, _NC))

    back_all = jnp.stack([
        jnp.pad(src_back_indices, (0, _NBACK_PAD - _NBACK),
                constant_values=_NPAD - 1),
        jnp.pad(ref_back_indices, (0, _NBACK_PAD - _NBACK),
                constant_values=_NPAD - 1)])
    gt_all = jnp.stack([gt_node_corr_indices[:, 1],
                        gt_node_corr_indices[:, 0]])
    zeros_all = jnp.zeros((2, _NPAD), jnp.int32)
    mask, gmask = _sc_masks(idx_all, back_all, gt_all, zeros_all)

    args = (
        mask[0].reshape(_NC, 1), mask[1].reshape(_NC, 1),
        gmask[0].reshape(_NGT, 1), gmask[1].reshape(_NGT, 1),
        ref_node_corr_indices.reshape(_NES, 1),
        src_node_corr_indices.reshape(_NES, 1),
        ref_node_corr_indices.reshape(1, _NES),
        src_node_corr_indices.reshape(1, _NES),
        gt_node_corr_indices[:, 0].reshape(_NGT, 1),
        gt_node_corr_indices[:, 1].reshape(_NGT, 1),
        gt_node_corr_indices[:, 0].reshape(1, _NGT),
        gt_node_corr_indices[:, 1].reshape(1, _NGT),
        gt_node_corr_overlaps.reshape(_NGT, 1).astype(f32),
        gt_node_corr_overlaps.reshape(1, _NGT).astype(f32),
    )
    out = _logic(args)
    loss = out[0, 0]
    loss1 = out[0, 1]
    loss2 = out[0, 2]
    return (loss, loss1, loss2)
